# baseline (speedup 1.0000x reference)
import functools

import jax
import jax.numpy as jnp
from jax import lax
from jax.experimental import pallas as pl
from jax.experimental.pallas import tpu as pltpu
from jax.experimental.pallas import tpu_sc as plsc

_NUM_CORES = 2
_NUM_SUBCORES = 16
_NUM_WORKERS = _NUM_CORES * _NUM_SUBCORES
_CHUNK = 128


def _dual_gather(num_chunks, user_input, game_input, user_table, game_table):
    b_per_w = num_chunks * _CHUNK
    batch = b_per_w * _NUM_WORKERS
    dim = user_table.shape[1]
    mesh = plsc.VectorSubcoreMesh(core_axis_name="c", subcore_axis_name="s")

    @functools.partial(
        pl.kernel,
        mesh=mesh,
        compiler_params=pltpu.CompilerParams(use_tc_tiling_on_sc=False),
        out_type=[
            jax.ShapeDtypeStruct((batch, dim), jnp.float32),
            jax.ShapeDtypeStruct((batch, dim), jnp.float32),
        ],
        scratch_types=[
            pltpu.VMEM((num_chunks, _CHUNK), jnp.int32),
            pltpu.VMEM((b_per_w, dim), jnp.float32),
            pltpu.VMEM((num_chunks, _CHUNK), jnp.int32),
            pltpu.VMEM((b_per_w, dim), jnp.float32),
            pltpu.SemaphoreType.DMA,
        ],
    )
    def dual_gather(uidx_hbm, gidx_hbm, utab_hbm, gtab_hbm, uout_hbm,
                    gout_hbm, uidx_v, urows_v, gidx_v, grows_v, sem):
        wid = lax.axis_index("s") * _NUM_CORES + lax.axis_index("c")
        base = wid * b_per_w
        pltpu.sync_copy(uidx_hbm.at[wid], uidx_v)
        pltpu.sync_copy(gidx_hbm.at[wid], gidx_v)
        copies = []
        for j in range(num_chunks):
            copies.append(pltpu.async_copy(
                utab_hbm.at[uidx_v.at[j]],
                urows_v.at[pl.ds(j * _CHUNK, _CHUNK)], sem))
            copies.append(pltpu.async_copy(
                gtab_hbm.at[gidx_v.at[j]],
                grows_v.at[pl.ds(j * _CHUNK, _CHUNK)], sem))
        for c in copies:
            c.wait()
        pltpu.sync_copy(urows_v, uout_hbm.at[pl.ds(base, b_per_w)])
        pltpu.sync_copy(grows_v, gout_hbm.at[pl.ds(base, b_per_w)])

    uidx = user_input.reshape(_NUM_WORKERS, num_chunks, _CHUNK)
    gidx = game_input.reshape(_NUM_WORKERS, num_chunks, _CHUNK)
    return dual_gather(uidx, gidx, user_table, game_table)


def kernel(user_input, game_input, user_table, game_table):
    batch = user_input.shape[0]
    assert batch % (_NUM_WORKERS * _CHUNK) == 0
    num_chunks = batch // (_NUM_WORKERS * _CHUNK)
    user_emb, game_emb = _dual_gather(
        num_chunks, user_input, game_input, user_table, game_table)
    return (user_emb, game_emb)
